# baseline (speedup 1.0000x reference)
import functools

import jax
import jax.numpy as jnp
from jax import lax
from jax.experimental import pallas as pl
from jax.experimental.pallas import tpu as pltpu
from jax.experimental.pallas import tpu_sc as plsc

_NUM_EXPERTS = 16
_TOP_K = 2
_N_TOKENS = 16384
_S = _N_TOKENS * _TOP_K
_LANES = 16
_NW = 32
_PER_TILE = _S // _NW
_PER_LANE = _PER_TILE // _LANES
_NCHUNK = _NW * _LANES


def _mesh():
    return plsc.VectorSubcoreMesh(core_axis_name="c", subcore_axis_name="s")


_SC_PARAMS = pltpu.CompilerParams(needs_layout_passes=False)


@functools.partial(
    pl.kernel,
    out_type=(
        jax.ShapeDtypeStruct((_NCHUNK * _NUM_EXPERTS,), jnp.int32),
        jax.ShapeDtypeStruct((_NW * _NUM_EXPERTS,), jnp.int32),
    ),
    mesh=_mesh(),
    compiler_params=_SC_PARAMS,
    scratch_types=[
        pltpu.VMEM((_PER_TILE,), jnp.int32),
        pltpu.VMEM((_LANES * _NUM_EXPERTS,), jnp.int32),
        pltpu.VMEM((_NUM_EXPERTS,), jnp.int32),
    ],
)
def _phase1(sel_hbm, h_hbm, ht_hbm, sel_v, cnt_v, tot_v):
    w = lax.axis_index("c") * _LANES + lax.axis_index("s")
    base = w * _PER_TILE
    pltpu.sync_copy(sel_hbm.at[pl.ds(base, _PER_TILE)], sel_v)

    lane = jnp.arange(_LANES, dtype=jnp.int32)
    lane16 = lane * _NUM_EXPERTS
    zero = jnp.zeros((_LANES,), jnp.int32)
    for l in range(_LANES):
        cnt_v[pl.ds(l * _NUM_EXPERTS, _NUM_EXPERTS)] = zero

    def body(t, carry):
        idx = lane * _PER_LANE + t
        e = plsc.load_gather(sel_v, [idx])
        a = lane16 + e
        c = plsc.load_gather(cnt_v, [a])
        plsc.store_scatter(cnt_v, [a], c + 1)
        return carry

    lax.fori_loop(0, _PER_LANE, body, 0)

    tot = zero
    for l in range(_LANES):
        tot = tot + cnt_v[pl.ds(l * _NUM_EXPERTS, _NUM_EXPERTS)]
    tot_v[...] = tot

    pltpu.sync_copy(cnt_v, h_hbm.at[pl.ds(w * _LANES * _NUM_EXPERTS,
                                          _LANES * _NUM_EXPERTS)])
    pltpu.sync_copy(tot_v, ht_hbm.at[pl.ds(w * _NUM_EXPERTS, _NUM_EXPERTS)])


@functools.partial(
    pl.kernel,
    out_type=(
        jax.ShapeDtypeStruct((_S,), jnp.float32),
        jax.ShapeDtypeStruct((_S,), jnp.int32),
        jax.ShapeDtypeStruct((_NUM_EXPERTS,), jnp.float32),
    ),
    mesh=_mesh(),
    compiler_params=_SC_PARAMS,
    scratch_types=[
        pltpu.VMEM((_PER_TILE,), jnp.int32),
        pltpu.VMEM((_PER_TILE,), jnp.float32),
        pltpu.VMEM((_PER_TILE,), jnp.int32),
        pltpu.VMEM((8, 128), jnp.int32),
        pltpu.VMEM((_LANES * _NUM_EXPERTS,), jnp.int32),
        pltpu.VMEM((_NW * _NUM_EXPERTS,), jnp.int32),
        pltpu.VMEM((_LANES * _NUM_EXPERTS,), jnp.int32),
        pltpu.VMEM((_NUM_EXPERTS,), jnp.float32),
        pltpu.SemaphoreType.DMA,
    ],
)
def _phase2(sel_hbm, sc_hbm, h_hbm, ht_hbm,
            out_sc, out_tok, out_cnt,
            sel_v, sc_v, tok_v, pos_v, cnt_v, ht_v, hmine_v, cntf_v, sem):
    w = lax.axis_index("c") * _LANES + lax.axis_index("s")
    base = w * _PER_TILE

    pltpu.sync_copy(sel_hbm.at[pl.ds(base, _PER_TILE)], sel_v)
    pltpu.sync_copy(sc_hbm.at[pl.ds(base, _PER_TILE)], sc_v)
    pltpu.sync_copy(ht_hbm, ht_v)
    pltpu.sync_copy(h_hbm.at[pl.ds(w * _LANES * _NUM_EXPERTS,
                                   _LANES * _NUM_EXPERTS)], hmine_v)

    lane = jnp.arange(_LANES, dtype=jnp.int32)
    lane16 = lane * _NUM_EXPERTS
    zero = jnp.zeros((_LANES,), jnp.int32)

    accA = zero
    accT = zero
    for wp in range(_NW):
        row = ht_v[pl.ds(wp * _NUM_EXPERTS, _NUM_EXPERTS)]
        accA = accA + jnp.where(wp < w, row, zero)
        accT = accT + row

    g = plsc.cumsum(accT) - accT

    run = accA
    for l in range(_LANES):
        cnt_v[pl.ds(l * _NUM_EXPERTS, _NUM_EXPERTS)] = g + run
        run = run + hmine_v[pl.ds(l * _NUM_EXPERTS, _NUM_EXPERTS)]

    def tok_body(j, carry):
        v = base + j * _LANES + lane
        tok_v[pl.ds(j * _LANES, _LANES)] = lax.shift_right_logical(v, 1)
        return carry

    lax.fori_loop(0, _PER_TILE // _LANES, tok_body, 0)

    def body(t, carry):
        idx = lane * _PER_LANE + t
        e = plsc.load_gather(sel_v, [idx])
        a = lane16 + e
        c = plsc.load_gather(cnt_v, [a])
        plsc.store_scatter(cnt_v, [a], c + 1)
        plsc.store_scatter(
            pos_v,
            [lax.shift_right_logical(idx, 7), lax.bitwise_and(idx, 127)],
            c,
        )
        return carry

    lax.fori_loop(0, _PER_LANE, body, 0)

    descs = []
    for j in range(8):
        descs.append(pltpu.async_copy(
            sc_v.at[pl.ds(j * 128, 128)], out_sc.at[pos_v.at[j]], sem))
        descs.append(pltpu.async_copy(
            tok_v.at[pl.ds(j * 128, 128)], out_tok.at[pos_v.at[j]], sem))
    for d in descs:
        d.wait()

    @pl.when(w == 0)
    def _():
        cntf_v[...] = accT.astype(jnp.float32)
        pltpu.sync_copy(cntf_v, out_cnt)


def kernel(top_scores, selected_experts_indices):
    sel = selected_experts_indices.reshape(-1).astype(jnp.int32)
    scores = top_scores.reshape(-1).astype(jnp.float32)
    h, ht = _phase1(sel)
    scores_sorted, token_idx, counts = _phase2(sel, scores, h, ht)
    return scores_sorted, token_idx, counts


# speedup vs baseline: 2.1222x; 2.1222x over previous
import functools

import jax
import jax.numpy as jnp
from jax import lax
from jax.experimental import pallas as pl
from jax.experimental.pallas import tpu as pltpu
from jax.experimental.pallas import tpu_sc as plsc

_NUM_EXPERTS = 16
_TOP_K = 2
_N_TOKENS = 16384
_S = _N_TOKENS * _TOP_K
_LANES = 16
_NW = 32
_PER_TILE = _S // _NW
_PER_LANE = _PER_TILE // _LANES
_NCHUNK = _NW * _LANES
_PER_SC_SLICE = _S // _LANES

_SC_PARAMS = pltpu.CompilerParams(needs_layout_passes=False)


def _mesh():
    return plsc.VectorSubcoreMesh(core_axis_name="c", subcore_axis_name="s")


def _hist_body(sel_ref, h_ref, tb_ref, cnt_ref):
    s = sel_ref[...]
    for e in range(_NUM_EXPERTS):
        m = (s == float(e)).astype(jnp.float32)
        h_ref[:, e:e + 1] = jnp.sum(m, axis=1, keepdims=True)
    h = h_ref[...]

    r32 = lax.broadcasted_iota(jnp.int32, (_NW, _NCHUNK), 0)
    c512 = lax.broadcasted_iota(jnp.int32, (_NW, _NCHUNK), 1)
    p = (lax.shift_right_logical(c512, 4) == r32).astype(jnp.float32)
    ht = jnp.dot(p, h, preferred_element_type=jnp.float32,
                 precision=lax.Precision.HIGHEST)

    total = jnp.sum(h, axis=0, keepdims=True)
    cnt_ref[...] = total

    ru = lax.broadcasted_iota(jnp.int32, (_NUM_EXPERTS, _NUM_EXPERTS), 0)
    cu = lax.broadcasted_iota(jnp.int32, (_NUM_EXPERTS, _NUM_EXPERTS), 1)
    upper = (ru < cu).astype(jnp.float32)
    g = jnp.dot(total, upper, preferred_element_type=jnp.float32,
                 precision=lax.Precision.HIGHEST)

    rl = lax.broadcasted_iota(jnp.int32, (_NW, _NW), 0)
    cl = lax.broadcasted_iota(jnp.int32, (_NW, _NW), 1)
    lower = (cl < rl).astype(jnp.float32)
    tb_ref[...] = jnp.dot(lower, ht, preferred_element_type=jnp.float32,
                 precision=lax.Precision.HIGHEST) + g


_hist = pl.pallas_call(
    _hist_body,
    out_shape=(
        jax.ShapeDtypeStruct((_NCHUNK, _NUM_EXPERTS), jnp.float32),
        jax.ShapeDtypeStruct((_NW, _NUM_EXPERTS), jnp.float32),
        jax.ShapeDtypeStruct((1, _NUM_EXPERTS), jnp.float32),
    ),
)


@functools.partial(
    pl.kernel,
    out_type=(
        jax.ShapeDtypeStruct((_S,), jnp.float32),
        jax.ShapeDtypeStruct((_S,), jnp.int32),
        jax.ShapeDtypeStruct((_S,), jnp.float32),
        jax.ShapeDtypeStruct((_S,), jnp.int32),
    ),
    mesh=_mesh(),
    compiler_params=_SC_PARAMS,
    scratch_types=[
        pltpu.VMEM((_PER_TILE,), jnp.int32),
        pltpu.VMEM((_PER_TILE,), jnp.float32),
        pltpu.VMEM((_PER_TILE,), jnp.int32),
        pltpu.VMEM((8, 128), jnp.int32),
        pltpu.VMEM((_LANES * _NUM_EXPERTS,), jnp.int32),
        pltpu.VMEM((_LANES * _NUM_EXPERTS,), jnp.int32),
        pltpu.VMEM((_NUM_EXPERTS,), jnp.int32),
        pltpu.VMEM((_PER_SC_SLICE,), jnp.float32),
        pltpu.VMEM((_PER_SC_SLICE,), jnp.int32),
        pltpu.VMEM_SHARED((_S,), jnp.float32),
        pltpu.VMEM_SHARED((_S,), jnp.int32),
        pltpu.SemaphoreType.DMA,
    ],
)
def _reorder(sel_hbm, sc_hbm, h_hbm, tb_hbm,
             pa_sc, pa_tok, pb_sc, pb_tok,
             sel_v, sc_v, tok_v, pos_v, cnt_v, hmine_v, tbrow_v,
             zf_v, zi_v, sp_sc, sp_tok, sem):
    c = lax.axis_index("c")
    s = lax.axis_index("s")
    w = c * _LANES + s
    base = w * _PER_TILE

    lane = jnp.arange(_LANES, dtype=jnp.int32)
    lane16 = lane * _NUM_EXPERTS
    zf = jnp.zeros((_LANES,), jnp.float32)
    zi = jnp.zeros((_LANES,), jnp.int32)

    def zero_body(j, carry):
        zf_v[pl.ds(j * _LANES, _LANES)] = zf
        zi_v[pl.ds(j * _LANES, _LANES)] = zi
        return carry

    lax.fori_loop(0, _PER_SC_SLICE // _LANES, zero_body, 0)
    sl = pl.ds(s * _PER_SC_SLICE, _PER_SC_SLICE)
    pltpu.sync_copy(zf_v, sp_sc.at[sl])
    pltpu.sync_copy(zi_v, sp_tok.at[sl])

    pltpu.sync_copy(sel_hbm.at[pl.ds(base, _PER_TILE)], sel_v)
    pltpu.sync_copy(sc_hbm.at[pl.ds(base, _PER_TILE)], sc_v)
    pltpu.sync_copy(h_hbm.at[pl.ds(w * _LANES * _NUM_EXPERTS,
                                   _LANES * _NUM_EXPERTS)], hmine_v)
    pltpu.sync_copy(tb_hbm.at[pl.ds(w * _NUM_EXPERTS, _NUM_EXPERTS)], tbrow_v)

    run = tbrow_v[...]
    for l in range(_LANES):
        cnt_v[pl.ds(l * _NUM_EXPERTS, _NUM_EXPERTS)] = run
        run = run + hmine_v[pl.ds(l * _NUM_EXPERTS, _NUM_EXPERTS)]

    def tok_body(j, carry):
        v = base + j * _LANES + lane
        tok_v[pl.ds(j * _LANES, _LANES)] = lax.shift_right_logical(v, 1)
        return carry

    lax.fori_loop(0, _PER_TILE // _LANES, tok_body, 0)

    def body(t, carry):
        idx = lane * _PER_LANE + t
        e = plsc.load_gather(sel_v, [idx])
        a = lane16 + e
        cc = plsc.load_gather(cnt_v, [a])
        plsc.store_scatter(cnt_v, [a], cc + 1)
        plsc.store_scatter(
            pos_v,
            [lax.shift_right_logical(idx, 7), lax.bitwise_and(idx, 127)],
            cc,
        )
        return carry

    lax.fori_loop(0, _PER_LANE, body, 0)

    plsc.subcore_barrier()

    descs = []
    for j in range(8):
        pj = pos_v.at[j]
        descs.append(pltpu.async_copy(
            sc_v.at[pl.ds(j * 128, 128)], sp_sc.at[pj], sem, add=True))
        descs.append(pltpu.async_copy(
            tok_v.at[pl.ds(j * 128, 128)], sp_tok.at[pj], sem, add=True))
    for d in descs:
        d.wait()

    plsc.subcore_barrier()

    @pl.when(c == 0)
    def _():
        pltpu.sync_copy(sp_sc.at[sl], pa_sc.at[sl])
        pltpu.sync_copy(sp_tok.at[sl], pa_tok.at[sl])

    @pl.when(c == 1)
    def _():
        pltpu.sync_copy(sp_sc.at[sl], pb_sc.at[sl])
        pltpu.sync_copy(sp_tok.at[sl], pb_tok.at[sl])


def _merge_body(pa_sc, pb_sc, pa_tok, pb_tok, out_sc, out_tok):
    out_sc[...] = pa_sc[...] + pb_sc[...]
    out_tok[...] = pa_tok[...] + pb_tok[...]


_merge = pl.pallas_call(
    _merge_body,
    out_shape=(
        jax.ShapeDtypeStruct((_S // 128, 128), jnp.float32),
        jax.ShapeDtypeStruct((_S // 128, 128), jnp.int32),
    ),
)


def kernel(top_scores, selected_experts_indices):
    sel = selected_experts_indices.reshape(-1).astype(jnp.int32)
    scores = top_scores.reshape(-1).astype(jnp.float32)

    sel_f = sel.reshape(_NCHUNK, _PER_LANE).astype(jnp.float32)
    h_f, tb_f, cnt_f = _hist(sel_f)
    h = h_f.reshape(-1).astype(jnp.int32)
    tb = tb_f.reshape(-1).astype(jnp.int32)

    pa_sc, pa_tok, pb_sc, pb_tok = _reorder(sel, scores, h, tb)

    out_sc, out_tok = _merge(
        pa_sc.reshape(_S // 128, 128), pb_sc.reshape(_S // 128, 128),
        pa_tok.reshape(_S // 128, 128), pb_tok.reshape(_S // 128, 128))

    return out_sc.reshape(-1), out_tok.reshape(-1), cnt_f.reshape(-1)


# speedup vs baseline: 5.0733x; 2.3906x over previous
import functools

import jax
import jax.numpy as jnp
from jax import lax
from jax.experimental import pallas as pl
from jax.experimental.pallas import tpu as pltpu
from jax.experimental.pallas import tpu_sc as plsc

_NUM_EXPERTS = 16
_TOP_K = 2
_N_TOKENS = 16384
_S = _N_TOKENS * _TOP_K
_LANES = 16
_NW = 32
_PER_TILE = _S // _NW
_PER_LANE = _PER_TILE // _LANES
_PER_SC_SLICE = _S // _LANES

_SC_PARAMS = pltpu.CompilerParams(needs_layout_passes=False)


def _mesh():
    return plsc.VectorSubcoreMesh(core_axis_name="c", subcore_axis_name="s")


def _hist_body(x_ref, tb_ref, cnt_ref, rs_ref):
    x = x_ref[...]
    for e in range(_NUM_EXPERTS):
        m = (x == e).astype(jnp.float32)
        rs_ref[:, e:e + 1] = jnp.sum(m, axis=1, keepdims=True)
    rs = rs_ref[...]

    rw = lax.broadcasted_iota(jnp.int32, (_NW, 256), 0)
    cr = lax.broadcasted_iota(jnp.int32, (_NW, 256), 1)
    wmat = (lax.shift_right_logical(cr, 3) == rw).astype(jnp.float32)
    ht = jnp.dot(wmat, rs, preferred_element_type=jnp.float32)

    total = jnp.sum(ht, axis=0, keepdims=True)
    cnt_ref[...] = total

    ru = lax.broadcasted_iota(jnp.int32, (_NUM_EXPERTS, _NUM_EXPERTS), 0)
    cu = lax.broadcasted_iota(jnp.int32, (_NUM_EXPERTS, _NUM_EXPERTS), 1)
    upper = (ru < cu).astype(jnp.float32)
    g = jnp.dot(total, upper, preferred_element_type=jnp.float32,
                precision=lax.Precision.HIGHEST)

    rl = lax.broadcasted_iota(jnp.int32, (_NW, _NW), 0)
    cl = lax.broadcasted_iota(jnp.int32, (_NW, _NW), 1)
    lower = (cl < rl).astype(jnp.float32)
    tb = jnp.dot(lower, ht, preferred_element_type=jnp.float32,
                 precision=lax.Precision.HIGHEST) + g
    tb_ref[:, 0:_NUM_EXPERTS] = tb


_hist = pl.pallas_call(
    _hist_body,
    out_shape=(
        jax.ShapeDtypeStruct((_NW, 128), jnp.float32),
        jax.ShapeDtypeStruct((1, _NUM_EXPERTS), jnp.float32),
    ),
    scratch_shapes=[pltpu.VMEM((256, _NUM_EXPERTS), jnp.float32)],
)


@functools.partial(
    pl.kernel,
    out_type=(
        jax.ShapeDtypeStruct((_S,), jnp.float32),
        jax.ShapeDtypeStruct((_S,), jnp.int32),
        jax.ShapeDtypeStruct((_S,), jnp.float32),
        jax.ShapeDtypeStruct((_S,), jnp.int32),
    ),
    mesh=_mesh(),
    compiler_params=_SC_PARAMS,
    scratch_types=[
        pltpu.VMEM((_PER_TILE,), jnp.int32),
        pltpu.VMEM((_PER_TILE,), jnp.float32),
        pltpu.VMEM((_PER_TILE,), jnp.int32),
        pltpu.VMEM((8, 128), jnp.int32),
        pltpu.VMEM((_LANES * _NUM_EXPERTS,), jnp.int32),
        pltpu.VMEM((_NUM_EXPERTS,), jnp.float32),
        pltpu.VMEM((_PER_SC_SLICE,), jnp.float32),
        pltpu.VMEM((_PER_SC_SLICE,), jnp.int32),
        pltpu.VMEM_SHARED((_S,), jnp.float32),
        pltpu.VMEM_SHARED((_S,), jnp.int32),
        pltpu.SemaphoreType.DMA,
    ],
)
def _reorder(sel_hbm, sc_hbm, tb_hbm,
             pa_sc, pa_tok, pb_sc, pb_tok,
             sel_v, sc_v, tok_v, pos_v, cnt_v, tbrow_v,
             zf_v, zi_v, sp_sc, sp_tok, sem):
    c = lax.axis_index("c")
    s = lax.axis_index("s")
    w = c * _LANES + s
    base = w * _PER_TILE

    lane = jnp.arange(_LANES, dtype=jnp.int32)
    lane16 = lane * _NUM_EXPERTS
    zf = jnp.zeros((_LANES,), jnp.float32)
    zi = jnp.zeros((_LANES,), jnp.int32)

    def zero_body(j, carry):
        zf_v[pl.ds(j * _LANES, _LANES)] = zf
        zi_v[pl.ds(j * _LANES, _LANES)] = zi
        return carry

    lax.fori_loop(0, _PER_SC_SLICE // _LANES, zero_body, 0)
    sl = pl.ds(s * _PER_SC_SLICE, _PER_SC_SLICE)
    pltpu.sync_copy(zf_v, sp_sc.at[sl])
    pltpu.sync_copy(zi_v, sp_tok.at[sl])

    pltpu.sync_copy(sel_hbm.at[pl.ds(base, _PER_TILE)], sel_v)
    pltpu.sync_copy(sc_hbm.at[pl.ds(base, _PER_TILE)], sc_v)
    pltpu.sync_copy(tb_hbm.at[pl.ds(w * 128, _NUM_EXPERTS)], tbrow_v)

    def addrmap(i):
        return (
            lax.bitwise_and(i, jnp.int32(~255))
            | lax.shift_left(lax.bitwise_and(i, 1), 7)
            | lax.shift_right_logical(lax.bitwise_and(i, 255), 1)
        )

    for l in range(_LANES):
        cnt_v[pl.ds(l * _NUM_EXPERTS, _NUM_EXPERTS)] = zi

    def hist_body(t, carry):
        ad = addrmap(lane * _PER_LANE + t)
        e = plsc.load_gather(sel_v, [ad])
        a = lane16 + e
        cc = plsc.load_gather(cnt_v, [a])
        plsc.store_scatter(cnt_v, [a], cc + 1)
        return carry

    lax.fori_loop(0, _PER_LANE, hist_body, 0)

    run = tbrow_v[...].astype(jnp.int32)
    for l in range(_LANES):
        csl = pl.ds(l * _NUM_EXPERTS, _NUM_EXPERTS)
        hl = cnt_v[csl]
        cnt_v[csl] = run
        run = run + hl

    def tok_body(j, carry):
        a = j * _LANES + lane
        inv = (
            lax.bitwise_and(a, jnp.int32(~255))
            | lax.shift_left(lax.bitwise_and(a, 127), 1)
            | lax.bitwise_and(lax.shift_right_logical(a, 7), 1)
        )
        tok_v[pl.ds(j * _LANES, _LANES)] = lax.shift_right_logical(base + inv, 1)
        return carry

    lax.fori_loop(0, _PER_TILE // _LANES, tok_body, 0)

    def body(t, carry):
        ad = addrmap(lane * _PER_LANE + t)
        e = plsc.load_gather(sel_v, [ad])
        a = lane16 + e
        cc = plsc.load_gather(cnt_v, [a])
        plsc.store_scatter(cnt_v, [a], cc + 1)
        plsc.store_scatter(
            pos_v,
            [lax.shift_right_logical(ad, 7), lax.bitwise_and(ad, 127)],
            cc,
        )
        return carry

    lax.fori_loop(0, _PER_LANE, body, 0)

    plsc.subcore_barrier()

    descs = []
    for j in range(8):
        pj = pos_v.at[j]
        descs.append(pltpu.async_copy(
            sc_v.at[pl.ds(j * 128, 128)], sp_sc.at[pj], sem, add=True))
        descs.append(pltpu.async_copy(
            tok_v.at[pl.ds(j * 128, 128)], sp_tok.at[pj], sem, add=True))
    for d in descs:
        d.wait()

    plsc.subcore_barrier()

    @pl.when(c == 0)
    def _():
        pltpu.sync_copy(sp_sc.at[sl], pa_sc.at[sl])
        pltpu.sync_copy(sp_tok.at[sl], pa_tok.at[sl])

    @pl.when(c == 1)
    def _():
        pltpu.sync_copy(sp_sc.at[sl], pb_sc.at[sl])
        pltpu.sync_copy(sp_tok.at[sl], pb_tok.at[sl])


def _merge_body(pa_sc, pb_sc, pa_tok, pb_tok, out_sc, out_tok):
    out_sc[...] = pa_sc[...] + pb_sc[...]
    out_tok[...] = pa_tok[...] + pb_tok[...]


_merge = pl.pallas_call(
    _merge_body,
    out_shape=(
        jax.ShapeDtypeStruct((_S // 128, 128), jnp.float32),
        jax.ShapeDtypeStruct((_S // 128, 128), jnp.int32),
    ),
)


def kernel(top_scores, selected_experts_indices):
    sel_lin = jnp.transpose(
        selected_experts_indices.astype(jnp.int32).reshape(128, 128, _TOP_K),
        (0, 2, 1)).reshape(_S)
    sc_lin = jnp.transpose(
        top_scores.reshape(128, 128, _TOP_K), (0, 2, 1)).reshape(_S)

    tb4, cnt = _hist(sel_lin.reshape(256, 128))
    tb_lin = tb4.reshape(-1)

    pa_sc, pa_tok, pb_sc, pb_tok = _reorder(sel_lin, sc_lin, tb_lin)

    out_sc, out_tok = _merge(
        pa_sc.reshape(_S // 128, 128), pb_sc.reshape(_S // 128, 128),
        pa_tok.reshape(_S // 128, 128), pb_tok.reshape(_S // 128, 128))

    return out_sc.reshape(-1), out_tok.reshape(-1), cnt.reshape(-1)


# speedup vs baseline: 5.5344x; 1.0909x over previous
import functools

import jax
import jax.numpy as jnp
from jax import lax
from jax.experimental import pallas as pl
from jax.experimental.pallas import tpu as pltpu
from jax.experimental.pallas import tpu_sc as plsc

_NUM_EXPERTS = 16
_TOP_K = 2
_N_TOKENS = 16384
_S = _N_TOKENS * _TOP_K
_LANES = 16
_NW = 32
_PER_TILE = _S // _NW
_PER_LANE = _PER_TILE // _LANES
_PER_SC_SLICE = _S // _LANES

_SC_PARAMS = pltpu.CompilerParams(needs_layout_passes=False)


def _mesh():
    return plsc.VectorSubcoreMesh(core_axis_name="c", subcore_axis_name="s")


def _hist_body(x_ref, tb_ref, cnt_ref, rs_ref):
    x = x_ref[...]
    for e in range(_NUM_EXPERTS):
        m = (x == e).astype(jnp.float32)
        rs_ref[:, e:e + 1] = jnp.sum(m, axis=1, keepdims=True)
    rs = rs_ref[...]

    rw = lax.broadcasted_iota(jnp.int32, (_NW, 256), 0)
    cr = lax.broadcasted_iota(jnp.int32, (_NW, 256), 1)
    wmat = (lax.shift_right_logical(cr, 3) == rw).astype(jnp.float32)
    ht = jnp.dot(wmat, rs, preferred_element_type=jnp.float32)

    total = jnp.sum(ht, axis=0, keepdims=True)
    cnt_ref[...] = total

    ru = lax.broadcasted_iota(jnp.int32, (_NUM_EXPERTS, _NUM_EXPERTS), 0)
    cu = lax.broadcasted_iota(jnp.int32, (_NUM_EXPERTS, _NUM_EXPERTS), 1)
    upper = (ru < cu).astype(jnp.float32)
    g = jnp.dot(total, upper, preferred_element_type=jnp.float32,
                precision=lax.Precision.HIGHEST)

    rl = lax.broadcasted_iota(jnp.int32, (_NW, _NW), 0)
    cl = lax.broadcasted_iota(jnp.int32, (_NW, _NW), 1)
    lower = (cl < rl).astype(jnp.float32)
    tb = jnp.dot(lower, ht, preferred_element_type=jnp.float32,
                 precision=lax.Precision.HIGHEST) + g
    tb_ref[:, 0:_NUM_EXPERTS] = tb


_hist = pl.pallas_call(
    _hist_body,
    out_shape=(
        jax.ShapeDtypeStruct((_NW, 128), jnp.float32),
        jax.ShapeDtypeStruct((1, _NUM_EXPERTS), jnp.float32),
    ),
    scratch_shapes=[pltpu.VMEM((256, _NUM_EXPERTS), jnp.float32)],
)


@functools.partial(
    pl.kernel,
    out_type=(
        jax.ShapeDtypeStruct((_S,), jnp.float32),
        jax.ShapeDtypeStruct((_S,), jnp.int32),
        jax.ShapeDtypeStruct((_S,), jnp.float32),
        jax.ShapeDtypeStruct((_S,), jnp.int32),
    ),
    mesh=_mesh(),
    compiler_params=_SC_PARAMS,
    scratch_types=[
        pltpu.VMEM((_PER_TILE,), jnp.int32),
        pltpu.VMEM((_PER_TILE,), jnp.float32),
        pltpu.VMEM((_PER_TILE,), jnp.int32),
        pltpu.VMEM((8, 128), jnp.int32),
        pltpu.VMEM((_LANES * _NUM_EXPERTS,), jnp.int32),
        pltpu.VMEM((_NUM_EXPERTS,), jnp.float32),
        pltpu.VMEM((_PER_SC_SLICE // 4,), jnp.float32),
        pltpu.VMEM((_PER_SC_SLICE // 4,), jnp.int32),
        pltpu.VMEM_SHARED((_S,), jnp.float32),
        pltpu.VMEM_SHARED((_S,), jnp.int32),
        pltpu.SemaphoreType.DMA,
        pltpu.SemaphoreType.DMA,
        pltpu.SemaphoreType.DMA,
        pltpu.SemaphoreType.DMA,
    ],
)
def _reorder(sel_hbm, sc_hbm, tb_hbm,
             pa_sc, pa_tok, pb_sc, pb_tok,
             sel_v, sc_v, tok_v, pos_v, cnt_v, tbrow_v,
             zf_v, zi_v, sp_sc, sp_tok, sem, sem_z, sem_in, sem_sc):
    c = lax.axis_index("c")
    s = lax.axis_index("s")
    w = c * _LANES + s
    base = w * _PER_TILE

    lane = jnp.arange(_LANES, dtype=jnp.int32)
    lane16 = lane * _NUM_EXPERTS
    zf = jnp.zeros((_LANES,), jnp.float32)
    zi = jnp.zeros((_LANES,), jnp.int32)

    d_sel = pltpu.async_copy(sel_hbm.at[pl.ds(base, _PER_TILE)], sel_v, sem_in)
    d_tb = pltpu.async_copy(tb_hbm.at[pl.ds(w * 128, _NUM_EXPERTS)],
                            tbrow_v, sem_in)
    d_sc = pltpu.async_copy(sc_hbm.at[pl.ds(base, _PER_TILE)], sc_v, sem_sc)

    _ZCH = _PER_SC_SLICE // 4

    def zero_body(j, carry):
        zf_v[pl.ds(j * _LANES, _LANES)] = zf
        zi_v[pl.ds(j * _LANES, _LANES)] = zi
        return carry

    lax.fori_loop(0, _ZCH // _LANES, zero_body, 0)
    zdescs = []
    for q in range(4):
        zsl = pl.ds(s * _PER_SC_SLICE + q * _ZCH, _ZCH)
        zdescs.append(pltpu.async_copy(zf_v, sp_sc.at[zsl], sem_z))
        zdescs.append(pltpu.async_copy(zi_v, sp_tok.at[zsl], sem_z))
    sl = pl.ds(s * _PER_SC_SLICE, _PER_SC_SLICE)

    d_sel.wait()
    d_tb.wait()

    def addrmap(i):
        return (
            lax.bitwise_and(i, jnp.int32(~255))
            | lax.shift_left(lax.bitwise_and(i, 1), 7)
            | lax.shift_right_logical(lax.bitwise_and(i, 255), 1)
        )

    for l in range(_LANES):
        cnt_v[pl.ds(l * _NUM_EXPERTS, _NUM_EXPERTS)] = zi

    def hist_body(t, carry):
        ad = addrmap(lane * _PER_LANE + t)
        e = plsc.load_gather(sel_v, [ad])
        a = lane16 + e
        cc = plsc.load_gather(cnt_v, [a])
        plsc.store_scatter(cnt_v, [a], cc + 1)
        return carry

    lax.fori_loop(0, _PER_LANE, hist_body, 0)

    run = tbrow_v[...].astype(jnp.int32)
    for l in range(_LANES):
        csl = pl.ds(l * _NUM_EXPERTS, _NUM_EXPERTS)
        hl = cnt_v[csl]
        cnt_v[csl] = run
        run = run + hl

    def tok_body(j, carry):
        a = j * _LANES + lane
        inv = (
            lax.bitwise_and(a, jnp.int32(~255))
            | lax.shift_left(lax.bitwise_and(a, 127), 1)
            | lax.bitwise_and(lax.shift_right_logical(a, 7), 1)
        )
        tok_v[pl.ds(j * _LANES, _LANES)] = lax.shift_right_logical(base + inv, 1)
        return carry

    lax.fori_loop(0, _PER_TILE // _LANES, tok_body, 0)

    def body(t, carry):
        ad = addrmap(lane * _PER_LANE + t)
        e = plsc.load_gather(sel_v, [ad])
        a = lane16 + e
        cc = plsc.load_gather(cnt_v, [a])
        plsc.store_scatter(cnt_v, [a], cc + 1)
        plsc.store_scatter(
            pos_v,
            [lax.shift_right_logical(ad, 7), lax.bitwise_and(ad, 127)],
            cc,
        )
        return carry

    lax.fori_loop(0, _PER_LANE, body, 0)

    for d in zdescs:
        d.wait()
    d_sc.wait()
    plsc.subcore_barrier()

    descs = []
    for j in range(8):
        pj = pos_v.at[j]
        descs.append(pltpu.async_copy(
            sc_v.at[pl.ds(j * 128, 128)], sp_sc.at[pj], sem, add=True))
        descs.append(pltpu.async_copy(
            tok_v.at[pl.ds(j * 128, 128)], sp_tok.at[pj], sem, add=True))
    for d in descs:
        d.wait()

    plsc.subcore_barrier()

    @pl.when(c == 0)
    def _():
        o1 = pltpu.async_copy(sp_sc.at[sl], pa_sc.at[sl], sem_in)
        o2 = pltpu.async_copy(sp_tok.at[sl], pa_tok.at[sl], sem_sc)
        o1.wait()
        o2.wait()

    @pl.when(c == 1)
    def _():
        o1 = pltpu.async_copy(sp_sc.at[sl], pb_sc.at[sl], sem_in)
        o2 = pltpu.async_copy(sp_tok.at[sl], pb_tok.at[sl], sem_sc)
        o1.wait()
        o2.wait()


def _merge_body(pa_sc, pb_sc, pa_tok, pb_tok, out_sc, out_tok):
    out_sc[...] = pa_sc[...] + pb_sc[...]
    out_tok[...] = pa_tok[...] + pb_tok[...]


_merge = pl.pallas_call(
    _merge_body,
    out_shape=(
        jax.ShapeDtypeStruct((_S // 128, 128), jnp.float32),
        jax.ShapeDtypeStruct((_S // 128, 128), jnp.int32),
    ),
)


def kernel(top_scores, selected_experts_indices):
    sel_lin = jnp.transpose(
        selected_experts_indices.astype(jnp.int32).reshape(128, 128, _TOP_K),
        (0, 2, 1)).reshape(_S)
    sc_lin = jnp.transpose(
        top_scores.reshape(128, 128, _TOP_K), (0, 2, 1)).reshape(_S)

    tb4, cnt = _hist(sel_lin.reshape(256, 128))
    tb_lin = tb4.reshape(-1)

    pa_sc, pa_tok, pb_sc, pb_tok = _reorder(sel_lin, sc_lin, tb_lin)

    out_sc, out_tok = _merge(
        pa_sc.reshape(_S // 128, 128), pb_sc.reshape(_S // 128, 128),
        pa_tok.reshape(_S // 128, 128), pb_tok.reshape(_S // 128, 128))

    return out_sc.reshape(-1), out_tok.reshape(-1), cnt.reshape(-1)
